# baseline (speedup 1.0000x reference)
import jax
import jax.numpy as jnp
from jax.experimental import pallas as pl
from jax.experimental.pallas import tpu as pltpu

INPUT_DIM = 832
HIDDEN1 = 256
HIDDEN2 = 32
BATCH = 16384
BM = 2048


def _mlp_block(x_ref, w1t_ref, b1_ref, w2t_ref, b2_ref, w3_ref, b3_ref, out_ref):
    x = x_ref[...]
    h = jnp.dot(x, w1t_ref[...], preferred_element_type=jnp.float32)
    h = jnp.maximum(h + b1_ref[...], 0.0)
    h = jnp.dot(h, w2t_ref[...], preferred_element_type=jnp.float32)
    h = jnp.maximum(h + b2_ref[...], 0.0)
    out = jnp.sum(h * w3_ref[...], axis=1, keepdims=True) + b3_ref[0, 0]
    out_ref[...] = out


def kernel(x, w1, b1, w2, b2, w3, b3):
    w1t = w1.T
    w2t = w2.T
    b1r = b1.reshape(1, HIDDEN1)
    b2r = b2.reshape(1, HIDDEN2)
    b3r = b3.reshape(1, 1)

    grid = (BATCH // BM,)
    const = lambda i: (0, 0)
    return pl.pallas_call(
        _mlp_block,
        grid=grid,
        in_specs=[
            pl.BlockSpec((BM, INPUT_DIM), lambda i: (i, 0)),
            pl.BlockSpec((INPUT_DIM, HIDDEN1), const),
            pl.BlockSpec((1, HIDDEN1), const),
            pl.BlockSpec((HIDDEN1, HIDDEN2), const),
            pl.BlockSpec((1, HIDDEN2), const),
            pl.BlockSpec((1, HIDDEN2), const),
            pl.BlockSpec((1, 1), const),
        ],
        out_specs=pl.BlockSpec((BM, 1), lambda i: (i, 0)),
        out_shape=jax.ShapeDtypeStruct((BATCH, 1), jnp.float32),
    )(x, w1t, b1r, w2t, b2r, w3, b3r)
